# Initial kernel scaffold; baseline (speedup 1.0000x reference)
import jax
import jax.numpy as jnp
from jax.experimental import pallas as pl


def kernel(inputs, table):
    raise NotImplementedError("write your pallas kernel here")


# speedup vs baseline: 8.1774x; 8.1774x over previous
import functools

import jax
import jax.numpy as jnp
from jax.experimental import pallas as pl

NUM_ROWS = 5
EMB = 64
BLOCK_B = 256


def _embed_block(idx_ref, tab_ref, out_ref):
    idx = idx_ref[...][..., None]
    tab = tab_ref[...]
    acc = jnp.broadcast_to(tab[0].reshape(1, 1, EMB), out_ref.shape)
    for r in range(1, NUM_ROWS):
        acc = jnp.where(idx == r, tab[r].reshape(1, 1, EMB), acc)
    out_ref[...] = acc


@jax.jit
def kernel(inputs, table):
    batch, seq = inputs.shape
    grid = (batch // BLOCK_B,)
    return pl.pallas_call(
        _embed_block,
        grid=grid,
        in_specs=[
            pl.BlockSpec((BLOCK_B, seq), lambda i: (i, 0)),
            pl.BlockSpec((NUM_ROWS, EMB), lambda i: (0, 0)),
        ],
        out_specs=pl.BlockSpec((BLOCK_B, seq, EMB), lambda i: (i, 0, 0)),
        out_shape=jax.ShapeDtypeStruct((batch, seq, EMB), table.dtype),
    )(inputs, table)
